# Initial kernel scaffold; baseline (speedup 1.0000x reference)
import jax
import jax.numpy as jnp
from jax.experimental import pallas as pl


def kernel(mean_pooling_vec, merge_text_vec, retrieved_visual_feature_embedding_cls, retrieved_textual_feature_embedding, retrieved_label_list, RRCP, W_text, a_text, W_img, a_img, label_table, W_out, b_out):
    raise NotImplementedError("write your pallas kernel here")


# speedup vs baseline: 4.7184x; 4.7184x over previous
import jax
import jax.numpy as jnp
from jax import lax
from jax.experimental import pallas as pl

_D = 768
_N = 200
_NL = 1000
_TH = 0.5
_NEG = -1e30


def _leaky(x):
    return jnp.where(x > 0, x, 0.2 * x)


def _gat_query_row(q, X, Wm, a_pair, phantom, valid_b):
    wa1 = jnp.sum(Wm * a_pair[0:1, :], axis=1)
    wa2 = jnp.sum(Wm * a_pair[1:2, :], axis=1)
    f0 = jnp.sum(q * wa1[None, :], axis=1, keepdims=True)
    gq = jnp.sum(q * wa2[None, :], axis=1, keepdims=True)
    g = jnp.sum(X * wa2[None, None, :], axis=2)
    e_q = _leaky(f0 + gq)
    e_n = _leaky(f0 + g)
    e_ph = _leaky(f0)
    e_n_m = jnp.where(valid_b, e_n, _NEG)
    m = jnp.maximum(jnp.max(e_n_m, axis=1, keepdims=True), e_q)
    m = jnp.maximum(m, jnp.where(phantom > 0, e_ph, _NEG))
    p_q = jnp.exp(e_q - m)
    p_n = jnp.where(valid_b, jnp.exp(e_n - m), 0.0)
    s = p_q + jnp.sum(p_n, axis=1, keepdims=True) + phantom * jnp.exp(e_ph - m)
    hagg = (p_q * q + jnp.sum(p_n[:, :, None] * X, axis=1)) / s
    return 0.5 * q + 0.5 * jnp.dot(hagg, Wm, preferred_element_type=jnp.float32)


def _body(qt_ref, qi_ref, vis_ref, txt_ref, rr_ref, lab_ref,
          Wt_ref, at_ref, Wi_ref, ai_ref, table_ref, Wo_ref, bo_ref, out_ref):
    rr = rr_ref[...]
    B = rr.shape[0]
    valid_b = rr > _TH
    rrz = jnp.where(rr < _TH, 0.0, rr)
    zero_row = jnp.max(rrz, axis=1, keepdims=True) == 0.0
    col0 = lax.broadcasted_iota(jnp.int32, rr.shape, 1) == 0
    rrz = jnp.where(col0 & zero_row, 1.0, rrz)
    w = rrz / (jnp.sum(rrz, axis=1, keepdims=True) + 1e-6)

    nvalid = jnp.sum(valid_b.astype(jnp.float32), axis=1, keepdims=True)
    is_last = lax.broadcasted_iota(jnp.int32, (B, 1), 0) == (B - 1)
    phantom = jnp.where(is_last, 0.0, _N - nvalid)

    ht0 = _gat_query_row(qt_ref[...], vis_ref[...], Wt_ref[...], at_ref[...],
                         jnp.zeros((B, 1), jnp.float32), valid_b)
    hi0 = _gat_query_row(qi_ref[...], txt_ref[...], Wi_ref[...], ai_ref[...],
                         phantom, valid_b)

    labs = lab_ref[...]
    iota_c = lax.broadcasted_iota(jnp.int32, (1, 1, _NL), 2)
    counts = jnp.zeros((B, _NL), jnp.float32)
    for j0 in range(0, _N, 8):
        lc = labs[:, j0:j0 + 8]
        wc = w[:, j0:j0 + 8]
        oh = lc[:, :, None] == iota_c
        counts = counts + jnp.sum(jnp.where(oh, wc[:, :, None], 0.0), axis=1)
    label_agg = jnp.dot(counts, table_ref[...], preferred_element_type=jnp.float32)

    fused = jnp.concatenate([ht0, hi0, label_agg], axis=1)
    out_ref[...] = jnp.dot(fused, Wo_ref[...], preferred_element_type=jnp.float32) + bo_ref[...]


def kernel(mean_pooling_vec, merge_text_vec, retrieved_visual_feature_embedding_cls,
           retrieved_textual_feature_embedding, retrieved_label_list, RRCP,
           W_text, a_text, W_img, a_img, label_table, W_out, b_out):
    vis = retrieved_visual_feature_embedding_cls[:, :_N, 0, :]
    txt = retrieved_textual_feature_embedding[:, :_N, 0, :]
    rr = RRCP[:, :_N]
    labs = retrieved_label_list[:, :_N]
    return pl.pallas_call(
        _body,
        out_shape=jax.ShapeDtypeStruct((mean_pooling_vec.shape[0], 2), jnp.float32),
    )(mean_pooling_vec, merge_text_vec, vis, txt, rr, labs,
      W_text, a_text.reshape(2, _D), W_img, a_img.reshape(2, _D),
      label_table, W_out, b_out.reshape(1, 2))
